# baseline (speedup 1.0000x reference)
import functools

import jax
import jax.numpy as jnp
from jax import lax
from jax.experimental import pallas as pl
from jax.experimental.pallas import tpu as pltpu
from jax.experimental.pallas import tpu_sc as plsc


def _vq_body(ze_ref, cb_ref, tok_ref, counts_ref, acc_ref, acc_smem):
    b = pl.program_id(0)
    nb = pl.num_programs(0)

    @pl.when(b == 0)
    def _init():
        counts_ref[...] = jnp.zeros_like(counts_ref)
        acc_smem[0] = 0.0

    zeb = ze_ref[0]
    cb = cb_ref[...]
    K = cb.shape[0]
    T = zeb.shape[1]

    neg2scores = jax.lax.dot_general(
        cb, zeb, (((1,), (0,)), ((), ())),
        preferred_element_type=jnp.float32) * -2.0
    cnorm = jnp.sum(cb * cb, axis=1, keepdims=True)
    zsq = jnp.sum(zeb * zeb, axis=0, keepdims=True)
    dists = (zsq + cnorm) + neg2scores

    mind = jnp.min(dists, axis=0)
    eqmask = dists == mind[None, :]
    eqf = jnp.where(eqmask, 1.0, 0.0)
    kiota = jax.lax.broadcasted_iota(jnp.int32, (K, T), 0)
    idx = jnp.min(jnp.where(eqmask, kiota, K), axis=0)

    tok_ref[0, 0, :] = idx

    ones_row = jnp.ones((1, T), dtype=jnp.float32)
    counts_ref[...] += jax.lax.dot_general(
        ones_row, eqf, (((1,), (1,)), ((), ())),
        preferred_element_type=jnp.float32)

    acc_smem[0] += jnp.sum(mind)

    @pl.when(b == nb - 1)
    def _fini():
        acc_ref[...] = acc_smem[0].reshape(1, 1)


def _tc_chunk(ze, codebook):
    Bc, D, Tp = ze.shape
    K = codebook.shape[0]

    return pl.pallas_call(
        _vq_body,
        grid=(Bc,),
        in_specs=[
            pl.BlockSpec((1, D, Tp), lambda b: (b, 0, 0)),
            pl.BlockSpec((K, D), lambda b: (0, 0)),
        ],
        out_specs=[
            pl.BlockSpec((1, 1, Tp), lambda b: (b, 0, 0)),
            pl.BlockSpec((1, K), lambda b: (0, 0)),
            pl.BlockSpec((1, 1), lambda b: (0, 0)),
        ],
        out_shape=[
            jax.ShapeDtypeStruct((Bc, 1, Tp), jnp.int32),
            jax.ShapeDtypeStruct((1, K), jnp.float32),
            jax.ShapeDtypeStruct((1, 1), jnp.float32),
        ],
        scratch_shapes=[
            pltpu.SMEM((1,), jnp.float32),
        ],
    )(ze, codebook)


def _scalars_body(counts_ref, accs_ref, commit_ref, perp_ref, *,
                  total_n, total_elems):
    p = jnp.sum(counts_ref[...], axis=0, keepdims=True) / total_n
    ent = jnp.sum(p * jnp.log(p + 1e-10))
    perp_ref[...] = jnp.exp(-ent).reshape(1, 1)
    commit_ref[...] = (0.25 * jnp.sum(accs_ref[...]) / total_elems
                       ).reshape(1, 1)


def _tc_scalars(counts_all, accs_all, total_n, total_elems):
    body = functools.partial(
        _scalars_body, total_n=total_n, total_elems=total_elems)
    return pl.pallas_call(
        body,
        out_shape=[
            jax.ShapeDtypeStruct((1, 1), jnp.float32),
            jax.ShapeDtypeStruct((1, 1), jnp.float32),
        ],
    )(counts_all, accs_all)


def _make_sc_gather(K, D, Bc, Tp):
    info = plsc.get_sparse_core_info()
    nc = info.num_cores
    nw = nc * info.num_subcores
    tokw = (Bc * Tp) // nw
    w_per_b = Tp // tokw
    n_chunks = tokw // 16

    @functools.partial(
        pl.kernel,
        out_type=jax.ShapeDtypeStruct((Bc, D, Tp), jnp.float32),
        mesh=plsc.VectorSubcoreMesh(core_axis_name="c", subcore_axis_name="s"),
        compiler_params=pltpu.CompilerParams(needs_layout_passes=False),
        scratch_types=[
            pltpu.VMEM((K * D,), jnp.float32),
            pltpu.VMEM((tokw,), jnp.int32),
            pltpu.VMEM((D, tokw), jnp.float32),
        ],
    )
    def sc_kernel(cb_hbm, idx_hbm, zq_hbm, cb_v, idx_v, out_v):
        wid = lax.axis_index("s") * nc + lax.axis_index("c")
        base = wid * tokw
        pltpu.sync_copy(cb_hbm, cb_v)
        pltpu.sync_copy(idx_hbm.at[pl.ds(base, tokw)], idx_v)

        @plsc.parallel_loop(0, n_chunks, unroll=4)
        def chunk(i):
            base16 = idx_v[pl.ds(i * 16, 16)] * D
            for d in range(D):
                out_v[d, pl.ds(i * 16, 16)] = plsc.load_gather(
                    cb_v, [base16 + d])

        b = wid // w_per_b
        t0 = (wid % w_per_b) * tokw
        pltpu.sync_copy(out_v, zq_hbm.at[b, :, pl.ds(t0, tokw)])

    return sc_kernel


def kernel(ze, codebook):
    B, D, Tp = ze.shape
    K = codebook.shape[0]
    C = 8
    Bc = B // C
    cb_flat = codebook.reshape(K * D)
    sc_gather = _make_sc_gather(K, D, Bc, Tp)

    toks, counts_l, accs, zqs = [], [], [], []
    for c in range(C):
        zc = lax.slice_in_dim(ze, c * Bc, (c + 1) * Bc, axis=0)
        tok3, cnts, acc = _tc_chunk(zc, codebook)
        zq_c = sc_gather(cb_flat, tok3.reshape(Bc * Tp))
        toks.append(tok3)
        counts_l.append(cnts)
        accs.append(acc)
        zqs.append(zq_c)

    commit, perp = _tc_scalars(
        jnp.concatenate(counts_l, axis=0),
        jnp.concatenate(accs, axis=0),
        float(B * Tp), float(B * D * Tp))
    zq = jnp.concatenate(zqs, axis=0)
    tok = jnp.concatenate(toks, axis=0).reshape(B, Tp)
    return (zq, tok, commit[0, 0], perp[0, 0])


# speedup vs baseline: 1.1799x; 1.1799x over previous
import functools

import jax
import jax.numpy as jnp
from jax import lax
from jax.experimental import pallas as pl
from jax.experimental.pallas import tpu as pltpu
from jax.experimental.pallas import tpu_sc as plsc


def _vq_body(ze_ref, cb_ref, tok_ref, counts_ref, acc_ref, acc_smem):
    b = pl.program_id(0)
    nb = pl.num_programs(0)

    @pl.when(b == 0)
    def _init():
        counts_ref[...] = jnp.zeros_like(counts_ref)
        acc_smem[0] = 0.0

    zeb = ze_ref[0]
    cb = cb_ref[...]
    K = cb.shape[0]
    T = zeb.shape[1]

    neg2scores = jax.lax.dot_general(
        cb, zeb, (((1,), (0,)), ((), ())),
        preferred_element_type=jnp.float32) * -2.0
    cnorm = jnp.sum(cb * cb, axis=1, keepdims=True)
    zsq = jnp.sum(zeb * zeb, axis=0, keepdims=True)
    dists = (zsq + cnorm) + neg2scores

    mind = jnp.min(dists, axis=0)
    eqmask = dists == mind[None, :]
    eqf = jnp.where(eqmask, 1.0, 0.0)
    kiota = jax.lax.broadcasted_iota(jnp.int32, (K, T), 0)
    idx = jnp.min(jnp.where(eqmask, kiota, K), axis=0)

    tok_ref[0, 0, :] = idx

    ones_row = jnp.ones((1, T), dtype=jnp.float32)
    counts_ref[...] += jax.lax.dot_general(
        ones_row, eqf, (((1,), (1,)), ((), ())),
        preferred_element_type=jnp.float32)

    acc_smem[0] += jnp.sum(mind)

    @pl.when(b == nb - 1)
    def _fini():
        acc_ref[...] = acc_smem[0].reshape(1, 1)


def _tc_chunk(ze, codebook):
    Bc, D, Tp = ze.shape
    K = codebook.shape[0]

    return pl.pallas_call(
        _vq_body,
        grid=(Bc,),
        in_specs=[
            pl.BlockSpec((1, D, Tp), lambda b: (b, 0, 0)),
            pl.BlockSpec((K, D), lambda b: (0, 0)),
        ],
        out_specs=[
            pl.BlockSpec((1, 1, Tp), lambda b: (b, 0, 0)),
            pl.BlockSpec((1, K), lambda b: (0, 0)),
            pl.BlockSpec((1, 1), lambda b: (0, 0)),
        ],
        out_shape=[
            jax.ShapeDtypeStruct((Bc, 1, Tp), jnp.int32),
            jax.ShapeDtypeStruct((1, K), jnp.float32),
            jax.ShapeDtypeStruct((1, 1), jnp.float32),
        ],
        scratch_shapes=[
            pltpu.SMEM((1,), jnp.float32),
        ],
    )(ze, codebook)


def _scalars_body(counts_ref, accs_ref, commit_ref, perp_ref, *,
                  total_n, total_elems):
    p = jnp.sum(counts_ref[...], axis=0, keepdims=True) / total_n
    ent = jnp.sum(p * jnp.log(p + 1e-10))
    perp_ref[...] = jnp.exp(-ent).reshape(1, 1)
    commit_ref[...] = (0.25 * jnp.sum(accs_ref[...]) / total_elems
                       ).reshape(1, 1)


def _tc_scalars(counts_all, accs_all, total_n, total_elems):
    body = functools.partial(
        _scalars_body, total_n=total_n, total_elems=total_elems)
    return pl.pallas_call(
        body,
        out_shape=[
            jax.ShapeDtypeStruct((1, 1), jnp.float32),
            jax.ShapeDtypeStruct((1, 1), jnp.float32),
        ],
    )(counts_all, accs_all)


def _make_sc_gather(K, D, Bc, Tp):
    info = plsc.get_sparse_core_info()
    nc = info.num_cores
    nw = nc * info.num_subcores
    tokw = (Bc * Tp) // nw
    w_per_b = Tp // tokw
    n_chunks = tokw // 16

    @functools.partial(
        pl.kernel,
        out_type=jax.ShapeDtypeStruct((Bc, D, Tp), jnp.float32),
        mesh=plsc.VectorSubcoreMesh(core_axis_name="c", subcore_axis_name="s"),
        compiler_params=pltpu.CompilerParams(needs_layout_passes=False),
        scratch_types=[
            pltpu.VMEM((K * D,), jnp.float32),
            pltpu.VMEM((tokw,), jnp.int32),
            pltpu.VMEM((D, tokw), jnp.float32),
        ],
    )
    def sc_kernel(cb_hbm, idx_hbm, zq_hbm, cb_v, idx_v, out_v):
        wid = lax.axis_index("s") * nc + lax.axis_index("c")
        base = wid * tokw
        pltpu.sync_copy(cb_hbm, cb_v)
        pltpu.sync_copy(idx_hbm.at[pl.ds(base, tokw)], idx_v)

        @plsc.parallel_loop(0, n_chunks, unroll=1)
        def chunk(i):
            base16 = idx_v[pl.ds(i * 16, 16)] * D
            for d in range(D):
                out_v[d, pl.ds(i * 16, 16)] = plsc.load_gather(
                    cb_v, [base16 + d])

        b = wid // w_per_b
        t0 = (wid % w_per_b) * tokw
        pltpu.sync_copy(out_v, zq_hbm.at[b, :, pl.ds(t0, tokw)])

    return sc_kernel


def kernel(ze, codebook):
    B, D, Tp = ze.shape
    K = codebook.shape[0]
    C = 4
    Bc = B // C
    cb_flat = codebook.reshape(K * D)
    sc_gather = _make_sc_gather(K, D, Bc, Tp)

    toks, counts_l, accs, zqs = [], [], [], []
    for c in range(C):
        zc = lax.slice_in_dim(ze, c * Bc, (c + 1) * Bc, axis=0)
        tok3, cnts, acc = _tc_chunk(zc, codebook)
        zq_c = sc_gather(cb_flat, tok3.reshape(Bc * Tp))
        toks.append(tok3)
        counts_l.append(cnts)
        accs.append(acc)
        zqs.append(zq_c)

    commit, perp = _tc_scalars(
        jnp.concatenate(counts_l, axis=0),
        jnp.concatenate(accs, axis=0),
        float(B * Tp), float(B * D * Tp))
    zq = jnp.concatenate(zqs, axis=0)
    tok = jnp.concatenate(toks, axis=0).reshape(B, Tp)
    return (zq, tok, commit[0, 0], perp[0, 0])
